# baseline (speedup 1.0000x reference)
import jax
import jax.numpy as jnp
from jax import lax
from jax.experimental import pallas as pl
from jax.experimental.pallas import tpu as pltpu

_B, _S, _D, _P = 4, 4096, 2048, 16
_NCHUNK = 8
_ROWS = _B * _S


def _dma_patch_body(mask_ref, pos_ref, chg_ref, x_ref, o_ref,
                    row_in, row_out, bulk_sem, row_sem):
    chunk = _ROWS // _NCHUNK
    bulk = [
        pltpu.make_async_copy(
            x_ref.at[pl.ds(i * chunk, chunk)],
            o_ref.at[pl.ds(i * chunk, chunk)],
            bulk_sem,
        )
        for i in range(_NCHUNK)
    ]
    for cp in bulk:
        cp.start()

    gathers = []
    for b in range(_B):
        r = b * _S + mask_ref[b]
        cp = pltpu.make_async_copy(
            x_ref.at[pl.ds(r, 1)], row_in.at[pl.ds(b, 1)], row_sem
        )
        cp.start()
        gathers.append(cp)
    for cp in gathers:
        cp.wait()

    d_iota = lax.broadcasted_iota(jnp.int32, (1, _D), 1)
    for b in range(_B):
        row = row_in[pl.ds(b, 1), :]
        delta_row = jnp.zeros((1, _D), jnp.float32)
        for p in range(_P):
            onehot = d_iota == pos_ref[b, p]
            val_p = jnp.sum(jnp.where(onehot, row, 0.0))
            delta_row = delta_row + jnp.where(
                onehot, chg_ref[b, p] * jnp.sign(val_p), 0.0
            )
        row_out[pl.ds(b, 1), :] = row + delta_row

    for cp in bulk:
        cp.wait()

    scatters = []
    for b in range(_B):
        r = b * _S + mask_ref[b]
        cp = pltpu.make_async_copy(
            row_out.at[pl.ds(b, 1)], o_ref.at[pl.ds(r, 1)], row_sem
        )
        cp.start()
        scatters.append(cp)
    for cp in scatters:
        cp.wait()


def kernel(x, mask_idxs, pos_positions, pos_changes):
    xf = x.reshape(_ROWS, _D)
    out = pl.pallas_call(
        _dma_patch_body,
        in_specs=[
            pl.BlockSpec(memory_space=pltpu.SMEM),
            pl.BlockSpec(memory_space=pltpu.SMEM),
            pl.BlockSpec(memory_space=pltpu.SMEM),
            pl.BlockSpec(memory_space=pl.ANY),
        ],
        out_specs=pl.BlockSpec(memory_space=pl.ANY),
        out_shape=jax.ShapeDtypeStruct((_ROWS, _D), jnp.float32),
        scratch_shapes=[
            pltpu.VMEM((_B, _D), jnp.float32),
            pltpu.VMEM((_B, _D), jnp.float32),
            pltpu.SemaphoreType.DMA,
            pltpu.SemaphoreType.DMA,
        ],
    )(mask_idxs, pos_positions, pos_changes, xf)
    return out.reshape(_B, _S, _D)


# speedup vs baseline: 47.6736x; 47.6736x over previous
import jax
import jax.numpy as jnp
from jax import lax
from jax.experimental import pallas as pl
from jax.experimental.pallas import tpu as pltpu

_B, _S, _D, _P = 4, 4096, 2048, 16
_ROWS = _B * _S
_N = 4
_C = 1024
_LA = 2


def _ring_body(mask_ref, pos_ref, chg_ref, x_ref, o_ref,
               buf, row_in, row_out, in_sems, out_sems, row_sem):
    nch = _ROWS // _C
    ins, outs = {}, {}

    def start_in(j):
        cp = pltpu.make_async_copy(
            x_ref.at[pl.ds(j * _C, _C)], buf.at[j % _N], in_sems.at[j % _N]
        )
        cp.start()
        ins[j] = cp

    def start_out(i):
        cp = pltpu.make_async_copy(
            buf.at[i % _N], o_ref.at[pl.ds(i * _C, _C)], out_sems.at[i % _N]
        )
        cp.start()
        outs[i] = cp

    row_cps = []
    for b in range(_B):
        r = b * _S + mask_ref[b]
        cp = pltpu.make_async_copy(
            x_ref.at[pl.ds(r, 1)], row_in.at[pl.ds(b, 1)], row_sem
        )
        cp.start()
        row_cps.append(cp)

    for j in range(min(_LA, nch)):
        start_in(j)
    for i in range(nch):
        j = i + _LA
        if j < nch:
            if j >= _N:
                outs[j - _N].wait()
            start_in(j)
        ins[i].wait()
        start_out(i)
    for i in range(max(0, nch - _N), nch):
        outs[i].wait()

    for cp in row_cps:
        cp.wait()
    d_iota = lax.broadcasted_iota(jnp.int32, (1, _D), 1)
    for b in range(_B):
        row = row_in[pl.ds(b, 1), :]
        delta_row = jnp.zeros((1, _D), jnp.float32)
        for p in range(_P):
            onehot = d_iota == pos_ref[b, p]
            val_p = jnp.sum(jnp.where(onehot, row, 0.0))
            delta_row = delta_row + jnp.where(
                onehot, chg_ref[b, p] * jnp.sign(val_p), 0.0
            )
        row_out[pl.ds(b, 1), :] = row + delta_row

    scatters = []
    for b in range(_B):
        r = b * _S + mask_ref[b]
        cp = pltpu.make_async_copy(
            row_out.at[pl.ds(b, 1)], o_ref.at[pl.ds(r, 1)], row_sem
        )
        cp.start()
        scatters.append(cp)
    for cp in scatters:
        cp.wait()


def kernel(x, mask_idxs, pos_positions, pos_changes):
    xf = x.reshape(_ROWS, _D)
    out = pl.pallas_call(
        _ring_body,
        in_specs=[
            pl.BlockSpec(memory_space=pltpu.SMEM),
            pl.BlockSpec(memory_space=pltpu.SMEM),
            pl.BlockSpec(memory_space=pltpu.SMEM),
            pl.BlockSpec(memory_space=pl.ANY),
        ],
        out_specs=pl.BlockSpec(memory_space=pl.ANY),
        out_shape=jax.ShapeDtypeStruct((_ROWS, _D), jnp.float32),
        scratch_shapes=[
            pltpu.VMEM((_N, _C, _D), jnp.float32),
            pltpu.VMEM((_B, _D), jnp.float32),
            pltpu.VMEM((_B, _D), jnp.float32),
            pltpu.SemaphoreType.DMA((_N,)),
            pltpu.SemaphoreType.DMA((_N,)),
            pltpu.SemaphoreType.DMA,
        ],
    )(mask_idxs, pos_positions, pos_changes, xf)
    return out.reshape(_B, _S, _D)
